# baseline (speedup 1.0000x reference)
import functools
import math

import jax
import jax.numpy as jnp
from jax import lax
from jax.experimental import pallas as pl
from jax.experimental.pallas import tpu as pltpu
from jax.experimental.pallas import tpu_sc as plsc

E = 16
K = 8
H = 1024
I = 4096
B = 16
IC = 1024
NC = I // IC

_SQRT_HALF = math.sqrt(0.5)


def _make_router():
    mesh = plsc.VectorSubcoreMesh(core_axis_name="c", subcore_axis_name="s", num_cores=1)

    @functools.partial(
        pl.kernel,
        mesh=mesh,
        out_type=jax.ShapeDtypeStruct((B, E), jnp.float32),
        compiler_params=pltpu.CompilerParams(needs_layout_passes=False),
        scratch_types=[
            pltpu.VMEM((H,), jnp.float32),
            pltpu.VMEM((H * E,), jnp.float32),
            pltpu.VMEM((E,), jnp.float32),
            pltpu.VMEM((E,), jnp.float32),
        ],
    )
    def router(x_hbm, wr_hbm, br_hbm, out_hbm, x_v, wr_v, br_v, row_v):
        wid = lax.axis_index("s")

        @pl.when(wid < B)
        def _():
            b = wid
            pltpu.sync_copy(x_hbm.at[b], x_v)
            pltpu.sync_copy(wr_hbm, wr_v)
            pltpu.sync_copy(br_hbm, br_v)

            def fma32(i, accs):
                a = list(accs)
                for half in range(2):
                    xv = x_v[pl.ds(i * 32 + half * 16, 16)]
                    for j in range(16):
                        f = i * 32 + half * 16 + j
                        a[j % 4] = a[j % 4] + xv[j] * wr_v[pl.ds(f * E, E)]
                return tuple(a)

            zeros = jnp.zeros((E,), jnp.float32)
            a0, a1, a2, a3 = lax.fori_loop(
                0, H // 32, fma32, (br_v[...], zeros, zeros, zeros))
            logits = (a0 + a1) + (a2 + a3)

            def bmax(v):
                return jnp.maximum(plsc.cummax(v),
                                   jnp.flip(plsc.cummax(jnp.flip(v))))

            def bsum(v):
                return jnp.cumsum(v) + jnp.flip(jnp.cumsum(jnp.flip(v))) - v

            m = bmax(logits)
            iota = lax.iota(jnp.int32, E)
            sk, _ = plsc.sort_key_val(logits, iota, descending=True)
            ninf = jnp.full((E,), -jnp.inf, jnp.float32)
            thr = bmax(jnp.where(iota >= K - 1, sk, ninf))
            gt = logits > thr
            cgt = bsum(gt.astype(jnp.int32))
            eq = (logits == thr).astype(jnp.int32)
            excl = jnp.cumsum(eq) - eq
            sel = gt | ((eq == 1) & (excl < K - cgt))
            ex = jnp.where(sel, jnp.exp(logits - m), 0.0)
            row_v[...] = ex / bsum(ex)
            pltpu.sync_copy(row_v, out_hbm.at[b])

    return router


_router_sc = _make_router()


def _moe_body(x_ref, comb_ref, w1_ref, b1_ref, w2_ref, b2_ref, out_ref):
    e = pl.program_id(0)
    c = pl.program_id(1)

    x = x_ref[...]
    h = jnp.dot(x, w1_ref[0], preferred_element_type=jnp.float32) + b1_ref[0]
    g = h * 0.5 * (1.0 + jax.lax.erf(h * _SQRT_HALF))
    p = jnp.dot(g, w2_ref[0], preferred_element_type=jnp.float32)

    lane = jax.lax.broadcasted_iota(jnp.int32, (B, E), 1)
    col = jnp.sum(jnp.where(lane == e, comb_ref[...], 0.0), axis=1,
                  keepdims=True)

    bterm = jnp.where(c == 0, b2_ref[0], 0.0)
    contrib = col * (p + bterm)
    out_ref[...] = jnp.where((e == 0) & (c == 0), 0.0, out_ref[...]) + contrib


@jax.jit
def kernel(hidden_states, W1, b1, W2, b2, Wr, br):
    Bt, S, _ = hidden_states.shape
    x = hidden_states.reshape(Bt * S, H)
    b1r = b1.reshape(E, 1, I)
    b2r = b2.reshape(E, 1, H)

    comb = _router_sc(x, Wr.reshape(-1), br)

    out = pl.pallas_call(
        _moe_body,
        grid=(E, NC),
        in_specs=[
            pl.BlockSpec((B, H), lambda e, c: (0, 0)),
            pl.BlockSpec((B, E), lambda e, c: (0, 0)),
            pl.BlockSpec((1, H, IC), lambda e, c: (e, 0, c)),
            pl.BlockSpec((1, 1, IC), lambda e, c: (e, 0, c)),
            pl.BlockSpec((1, IC, H), lambda e, c: (e, c, 0)),
            pl.BlockSpec((1, 1, H), lambda e, c: (e, 0, 0)),
        ],
        out_specs=pl.BlockSpec((B, H), lambda e, c: (0, 0)),
        out_shape=jax.ShapeDtypeStruct((B, H), jnp.float32),
    )(x, comb, W1, b1r, W2, b2r)

    return out.reshape(Bt, S, H)


# speedup vs baseline: 1.0111x; 1.0111x over previous
import functools
import math

import jax
import jax.numpy as jnp
from jax import lax
from jax.experimental import pallas as pl
from jax.experimental.pallas import tpu as pltpu
from jax.experimental.pallas import tpu_sc as plsc

E = 16
K = 8
H = 1024
I = 4096
B = 16
IC = 1024
NC = I // IC

_SQRT_HALF = math.sqrt(0.5)


def _make_router():
    mesh = plsc.VectorSubcoreMesh(core_axis_name="c", subcore_axis_name="s", num_cores=1)

    @functools.partial(
        pl.kernel,
        mesh=mesh,
        out_type=jax.ShapeDtypeStruct((B, E), jnp.float32),
        compiler_params=pltpu.CompilerParams(needs_layout_passes=False),
        scratch_types=[
            pltpu.VMEM((H,), jnp.float32),
            pltpu.VMEM((H * E,), jnp.float32),
            pltpu.VMEM((E,), jnp.float32),
            pltpu.VMEM((E,), jnp.float32),
            pltpu.SemaphoreType.DMA,
        ],
    )
    def router(x_hbm, wr_hbm, br_hbm, out_hbm, x_v, wr_v, br_v, row_v, sem):
        wid = lax.axis_index("s")

        @pl.when(wid < B)
        def _():
            b = wid
            nq = 4
            qw = (H // nq) * E
            copies = [
                pltpu.async_copy(wr_hbm.at[pl.ds(q * qw, qw)],
                                 wr_v.at[pl.ds(q * qw, qw)], sem)
                for q in range(nq)
            ]
            pltpu.sync_copy(x_hbm.at[b], x_v)
            pltpu.sync_copy(br_hbm, br_v)

            def fma32(i, accs):
                a = list(accs)
                for half in range(2):
                    xv = x_v[pl.ds(i * 32 + half * 16, 16)]
                    for j in range(16):
                        f = i * 32 + half * 16 + j
                        a[j % 4] = a[j % 4] + xv[j] * wr_v[pl.ds(f * E, E)]
                return tuple(a)

            zeros = jnp.zeros((E,), jnp.float32)
            accs = (br_v[...], zeros, zeros, zeros)
            per_q = (H // nq) // 32
            for q in range(nq):
                copies[q].wait()
                accs = lax.fori_loop(q * per_q, (q + 1) * per_q, fma32, accs)
            a0, a1, a2, a3 = accs
            logits = (a0 + a1) + (a2 + a3)

            def bmax(v):
                return jnp.maximum(plsc.cummax(v),
                                   jnp.flip(plsc.cummax(jnp.flip(v))))

            def bsum(v):
                return jnp.cumsum(v) + jnp.flip(jnp.cumsum(jnp.flip(v))) - v

            m = bmax(logits)
            iota = lax.iota(jnp.int32, E)
            sk, _ = plsc.sort_key_val(logits, iota, descending=True)
            ninf = jnp.full((E,), -jnp.inf, jnp.float32)
            thr = bmax(jnp.where(iota >= K - 1, sk, ninf))
            gt = logits > thr
            cgt = bsum(gt.astype(jnp.int32))
            eq = (logits == thr).astype(jnp.int32)
            excl = jnp.cumsum(eq) - eq
            sel = gt | ((eq == 1) & (excl < K - cgt))
            ex = jnp.where(sel, jnp.exp(logits - m), 0.0)
            row_v[...] = ex / bsum(ex)
            pltpu.sync_copy(row_v, out_hbm.at[b])

    return router


_router_sc = _make_router()


def _moe_body(x_ref, comb_ref, w1_ref, b1_ref, w2_ref, b2_ref, out_ref):
    e = pl.program_id(0)
    c = pl.program_id(1)

    x = x_ref[...]
    h = jnp.dot(x, w1_ref[0], preferred_element_type=jnp.float32) + b1_ref[0]
    g = h * 0.5 * (1.0 + jax.lax.erf(h * _SQRT_HALF))
    p = jnp.dot(g, w2_ref[0], preferred_element_type=jnp.float32)

    lane = jax.lax.broadcasted_iota(jnp.int32, (B, E), 1)
    col = jnp.sum(jnp.where(lane == e, comb_ref[...], 0.0), axis=1,
                  keepdims=True)

    bterm = jnp.where(c == 0, b2_ref[0], 0.0)
    contrib = col * (p + bterm)
    out_ref[...] = jnp.where((e == 0) & (c == 0), 0.0, out_ref[...]) + contrib


@jax.jit
def kernel(hidden_states, W1, b1, W2, b2, Wr, br):
    Bt, S, _ = hidden_states.shape
    x = hidden_states.reshape(Bt * S, H)
    b1r = b1.reshape(E, 1, I)
    b2r = b2.reshape(E, 1, H)

    comb = _router_sc(x, Wr.reshape(-1), br)

    out = pl.pallas_call(
        _moe_body,
        grid=(E, NC),
        in_specs=[
            pl.BlockSpec((B, H), lambda e, c: (0, 0)),
            pl.BlockSpec((B, E), lambda e, c: (0, 0)),
            pl.BlockSpec((1, H, IC), lambda e, c: (e, 0, c)),
            pl.BlockSpec((1, 1, IC), lambda e, c: (e, 0, c)),
            pl.BlockSpec((1, IC, H), lambda e, c: (e, c, 0)),
            pl.BlockSpec((1, 1, H), lambda e, c: (e, 0, 0)),
        ],
        out_specs=pl.BlockSpec((B, H), lambda e, c: (0, 0)),
        out_shape=jax.ShapeDtypeStruct((B, H), jnp.float32),
    )(x, comb, W1, b1r, W2, b2r)

    return out.reshape(Bt, S, H)


# speedup vs baseline: 1.0115x; 1.0004x over previous
import functools
import math

import jax
import jax.numpy as jnp
from jax import lax
from jax.experimental import pallas as pl
from jax.experimental.pallas import tpu as pltpu
from jax.experimental.pallas import tpu_sc as plsc

E = 16
K = 8
H = 1024
I = 4096
B = 16
IC = 1024
NC = I // IC

_SQRT_HALF = math.sqrt(0.5)


def _make_router():
    mesh = plsc.VectorSubcoreMesh(core_axis_name="c", subcore_axis_name="s", num_cores=1)

    @functools.partial(
        pl.kernel,
        mesh=mesh,
        out_type=jax.ShapeDtypeStruct((B, E), jnp.float32),
        compiler_params=pltpu.CompilerParams(needs_layout_passes=False),
        scratch_types=[
            pltpu.VMEM((H,), jnp.float32),
            pltpu.VMEM((H * E,), jnp.float32),
            pltpu.VMEM((E,), jnp.float32),
            pltpu.VMEM((E,), jnp.float32),
            pltpu.SemaphoreType.DMA,
        ],
    )
    def router(x_hbm, wr_hbm, br_hbm, out_hbm, x_v, wr_v, br_v, row_v, sem):
        wid = lax.axis_index("s")

        @pl.when(wid < B)
        def _():
            b = wid
            nq = 8
            qw = (H // nq) * E
            copies = [
                pltpu.async_copy(wr_hbm.at[pl.ds(q * qw, qw)],
                                 wr_v.at[pl.ds(q * qw, qw)], sem)
                for q in range(nq)
            ]
            pltpu.sync_copy(x_hbm.at[b], x_v)
            pltpu.sync_copy(br_hbm, br_v)

            def fma32(i, accs):
                a = list(accs)
                for half in range(2):
                    xv = x_v[pl.ds(i * 32 + half * 16, 16)]
                    for j in range(16):
                        f = i * 32 + half * 16 + j
                        a[j % 4] = a[j % 4] + xv[j] * wr_v[pl.ds(f * E, E)]
                return tuple(a)

            zeros = jnp.zeros((E,), jnp.float32)
            accs = (br_v[...], zeros, zeros, zeros)
            per_q = (H // nq) // 32
            for q in range(nq):
                copies[q].wait()
                accs = lax.fori_loop(q * per_q, (q + 1) * per_q, fma32, accs)
            a0, a1, a2, a3 = accs
            logits = (a0 + a1) + (a2 + a3)

            def bmax(v):
                return jnp.maximum(plsc.cummax(v),
                                   jnp.flip(plsc.cummax(jnp.flip(v))))

            def bsum(v):
                return jnp.cumsum(v) + jnp.flip(jnp.cumsum(jnp.flip(v))) - v

            m = bmax(logits)
            iota = lax.iota(jnp.int32, E)
            sk, _ = plsc.sort_key_val(logits, iota, descending=True)
            ninf = jnp.full((E,), -jnp.inf, jnp.float32)
            thr = bmax(jnp.where(iota >= K - 1, sk, ninf))
            gt = logits > thr
            cgt = bsum(gt.astype(jnp.int32))
            eq = (logits == thr).astype(jnp.int32)
            excl = jnp.cumsum(eq) - eq
            sel = gt | ((eq == 1) & (excl < K - cgt))
            ex = jnp.where(sel, jnp.exp(logits - m), 0.0)
            row_v[...] = ex / bsum(ex)
            pltpu.sync_copy(row_v, out_hbm.at[b])

    return router


_router_cache = []


def _get_router():
    if not _router_cache:
        _router_cache.append(_make_router())
    return _router_cache[0]


def _moe_body(x_ref, comb_ref, w1_ref, b1_ref, w2_ref, b2_ref, out_ref):
    e = pl.program_id(0)
    c = pl.program_id(1)

    x = x_ref[...]
    h = jnp.dot(x, w1_ref[0], preferred_element_type=jnp.float32) + b1_ref[0]
    g = h * 0.5 * (1.0 + jax.lax.erf(h * _SQRT_HALF))
    p = jnp.dot(g, w2_ref[0], preferred_element_type=jnp.float32)

    lane = jax.lax.broadcasted_iota(jnp.int32, (B, E), 1)
    col = jnp.sum(jnp.where(lane == e, comb_ref[...], 0.0), axis=1,
                  keepdims=True)

    bterm = jnp.where(c == 0, b2_ref[0], 0.0)
    contrib = col * (p + bterm)
    out_ref[...] = jnp.where((e == 0) & (c == 0), 0.0, out_ref[...]) + contrib


@jax.jit
def kernel(hidden_states, W1, b1, W2, b2, Wr, br):
    Bt, S, _ = hidden_states.shape
    x = hidden_states.reshape(Bt * S, H)
    b1r = b1.reshape(E, 1, I)
    b2r = b2.reshape(E, 1, H)

    comb = _get_router()(x, Wr.reshape(-1), br)

    out = pl.pallas_call(
        _moe_body,
        grid=(E, NC),
        in_specs=[
            pl.BlockSpec((B, H), lambda e, c: (0, 0)),
            pl.BlockSpec((B, E), lambda e, c: (0, 0)),
            pl.BlockSpec((1, H, IC), lambda e, c: (e, 0, c)),
            pl.BlockSpec((1, 1, IC), lambda e, c: (e, 0, c)),
            pl.BlockSpec((1, IC, H), lambda e, c: (e, c, 0)),
            pl.BlockSpec((1, 1, H), lambda e, c: (e, 0, 0)),
        ],
        out_specs=pl.BlockSpec((B, H), lambda e, c: (0, 0)),
        out_shape=jax.ShapeDtypeStruct((B, H), jnp.float32),
    )(x, comb, W1, b1r, W2, b2r)

    return out.reshape(Bt, S, H)
